# Initial kernel scaffold; baseline (speedup 1.0000x reference)
import jax
import jax.numpy as jnp
from jax.experimental import pallas as pl


def kernel(tokens, embedding, embedding2):
    raise NotImplementedError("write your pallas kernel here")


# speedup vs baseline: 4.3853x; 4.3853x over previous
import functools

import jax
import jax.numpy as jnp
from jax import lax
from jax.experimental import pallas as pl
from jax.experimental.pallas import tpu as pltpu, tpu_sc as plsc

NUM_TOKENS = 1000
N_EMBD = 64
BATCH = 4096
MAX_LEN = 200

_CHUNK = 128
_NC = 2
_NS = 16


def _sc_body(n_chunks, tok_hbm, emb1_hbm, emb2_hbm, out1_hbm, out2_hbm,
             idx_v, rows1_v, rows2_v, sem1, sem2):
    wid = lax.axis_index("s") * _NC + lax.axis_index("c")
    chunk_base = wid * n_chunks
    pltpu.sync_copy(tok_hbm.at[pl.ds(chunk_base, n_chunks)], idx_v)

    def step(j, carry):
        idx_row = idx_v.at[j]
        cp1 = pltpu.async_copy(emb1_hbm.at[idx_row], rows1_v, sem1)
        cp2 = pltpu.async_copy(emb2_hbm.at[idx_row], rows2_v, sem2)
        cp1.wait()
        cp2.wait()
        out_base = (chunk_base + j) * _CHUNK
        pltpu.sync_copy(rows1_v, out1_hbm.at[pl.ds(out_base, _CHUNK)])
        pltpu.sync_copy(rows2_v, out2_hbm.at[pl.ds(out_base, _CHUNK)])
        return carry

    lax.fori_loop(0, n_chunks, step, 0)


def kernel(tokens, embedding, embedding2):
    B, L = tokens.shape
    V, D = embedding.shape
    n_flat = B * L
    assert n_flat % _CHUNK == 0
    nw = _NC * _NS
    n_chunks_total = n_flat // _CHUNK
    assert n_chunks_total % nw == 0
    n_chunks = n_chunks_total // nw

    tok2d = tokens.astype(jnp.int32).reshape(n_chunks_total, _CHUNK)

    mesh = plsc.VectorSubcoreMesh(core_axis_name="c", subcore_axis_name="s")
    out_sds = jax.ShapeDtypeStruct((n_flat, D), jnp.float32)
    run = pl.kernel(
        functools.partial(_sc_body, n_chunks),
        mesh=mesh,
        out_type=[out_sds, out_sds],
        scratch_types=[
            pltpu.VMEM((n_chunks, _CHUNK), jnp.int32),
            pltpu.VMEM((_CHUNK, D), jnp.float32),
            pltpu.VMEM((_CHUNK, D), jnp.float32),
            pltpu.SemaphoreType.DMA,
            pltpu.SemaphoreType.DMA,
        ],
        compiler_params=pltpu.CompilerParams(use_tc_tiling_on_sc=False),
    )
    out1, out2 = run(tok2d, embedding, embedding2)
    return (out1.reshape(B, L, D), out2.reshape(B, L, D))


# speedup vs baseline: 4.4632x; 1.0178x over previous
import functools

import jax
import jax.numpy as jnp
from jax import lax
from jax.experimental import pallas as pl
from jax.experimental.pallas import tpu as pltpu, tpu_sc as plsc

NUM_TOKENS = 1000
N_EMBD = 64
BATCH = 4096
MAX_LEN = 200

_CHUNK = 128
_NC = 2
_NS = 16


_K = 2


def _sc_body(n_chunks, tok_hbm, emb1_hbm, emb2_hbm, out1_hbm, out2_hbm,
             idx_v, r1a, r2a, r1b, r2b, sga, sgb, ssa, ssb):
    wid = lax.axis_index("s") * _NC + lax.axis_index("c")
    chunk_base = wid * n_chunks
    nsc = n_chunks // _K
    rows = _K * _CHUNK
    pltpu.sync_copy(tok_hbm.at[pl.ds(chunk_base, n_chunks)], idx_v)

    def fire_gather(g, r1, r2, sem):
        for b in range(_K):
            row = idx_v.at[g * _K + b]
            pltpu.async_copy(emb1_hbm.at[row], r1.at[pl.ds(b * _CHUNK, _CHUNK)], sem)
            pltpu.async_copy(emb2_hbm.at[row], r2.at[pl.ds(b * _CHUNK, _CHUNK)], sem)

    def wait_gather(r1, r2, sem):
        pltpu.make_async_copy(out1_hbm.at[pl.ds(0, rows)], r1, sem).wait()
        pltpu.make_async_copy(out2_hbm.at[pl.ds(0, rows)], r2, sem).wait()

    def fire_scatter(g, r1, r2, sem):
        out_base = (chunk_base + g * _K) * _CHUNK
        pltpu.async_copy(r1, out1_hbm.at[pl.ds(out_base, rows)], sem)
        pltpu.async_copy(r2, out2_hbm.at[pl.ds(out_base, rows)], sem)

    def wait_scatter(r1, r2, sem):
        pltpu.make_async_copy(r1, out1_hbm.at[pl.ds(0, rows)], sem).wait()
        pltpu.make_async_copy(r2, out2_hbm.at[pl.ds(0, rows)], sem).wait()

    fire_gather(0, r1a, r2a, sga)

    def body(t, carry):
        g0 = 2 * t
        g1 = g0 + 1
        pl.when(t > 0)(lambda: wait_scatter(r1b, r2b, ssb))
        fire_gather(g1, r1b, r2b, sgb)
        wait_gather(r1a, r2a, sga)
        fire_scatter(g0, r1a, r2a, ssa)
        wait_scatter(r1a, r2a, ssa)
        pl.when(t < nsc // 2 - 1)(lambda: fire_gather(g0 + 2, r1a, r2a, sga))
        wait_gather(r1b, r2b, sgb)
        fire_scatter(g1, r1b, r2b, ssb)
        return carry

    lax.fori_loop(0, nsc // 2, body, 0)
    wait_scatter(r1b, r2b, ssb)


def kernel(tokens, embedding, embedding2):
    B, L = tokens.shape
    V, D = embedding.shape
    n_flat = B * L
    assert n_flat % _CHUNK == 0
    nw = _NC * _NS
    n_chunks_total = n_flat // _CHUNK
    assert n_chunks_total % nw == 0
    n_chunks = n_chunks_total // nw

    tok2d = tokens.astype(jnp.int32).reshape(n_chunks_total, _CHUNK)

    mesh = plsc.VectorSubcoreMesh(core_axis_name="c", subcore_axis_name="s")
    out_sds = jax.ShapeDtypeStruct((n_flat, D), jnp.float32)
    run = pl.kernel(
        functools.partial(_sc_body, n_chunks),
        mesh=mesh,
        out_type=[out_sds, out_sds],
        scratch_types=[
            pltpu.VMEM((n_chunks, _CHUNK), jnp.int32),
            pltpu.VMEM((_K * _CHUNK, D), jnp.float32),
            pltpu.VMEM((_K * _CHUNK, D), jnp.float32),
            pltpu.VMEM((_K * _CHUNK, D), jnp.float32),
            pltpu.VMEM((_K * _CHUNK, D), jnp.float32),
            pltpu.SemaphoreType.DMA,
            pltpu.SemaphoreType.DMA,
            pltpu.SemaphoreType.DMA,
            pltpu.SemaphoreType.DMA,
        ],
        compiler_params=pltpu.CompilerParams(use_tc_tiling_on_sc=False),
    )
    out1, out2 = run(tok2d, embedding, embedding2)
    return (out1.reshape(B, L, D), out2.reshape(B, L, D))


# speedup vs baseline: 6.1420x; 1.3762x over previous
import functools

import jax
import jax.numpy as jnp
from jax import lax
from jax.experimental import pallas as pl
from jax.experimental.pallas import tpu as pltpu, tpu_sc as plsc

_CHUNK = 128
_NC = 2
_NS = 16
_LANE = 16


def _sc_body(n_chunks, D, tok_hbm, cat_hbm, out1_hbm, out2_hbm,
             idx_v, ca, cb, o1a, o2a, o1b, o2b, sga, sgb, ssa, ssb):
    wid = lax.axis_index("s") * _NC + lax.axis_index("c")
    chunk_base = wid * n_chunks
    pltpu.sync_copy(tok_hbm.at[pl.ds(chunk_base, n_chunks)], idx_v)

    def fire_gather(g, cbuf, sem):
        pltpu.async_copy(cat_hbm.at[idx_v.at[g]], cbuf, sem)

    def wait_gather(cbuf, sem):
        pltpu.make_async_copy(out1_hbm.at[pl.ds(0, _CHUNK)], cbuf, sem).wait()

    def fire_scatter(g, o1, o2, sem):
        out_base = (chunk_base + g) * _CHUNK
        pltpu.async_copy(o1, out1_hbm.at[pl.ds(out_base, _CHUNK)], sem)
        pltpu.async_copy(o2, out2_hbm.at[pl.ds(out_base, _CHUNK)], sem)

    def wait_scatter(o1, o2, sem):
        pltpu.make_async_copy(o1, out1_hbm.at[pl.ds(0, _CHUNK)], sem).wait()
        pltpu.make_async_copy(o2, out2_hbm.at[pl.ds(0, _CHUNK)], sem).wait()

    nv = D // _LANE

    def deinterleave(cbuf, o1, o2):
        rows_per_iter = 8

        def rows(i, carry):
            r0 = i * rows_per_iter
            for dr in range(rows_per_iter):
                r = r0 + dr
                for c in range(nv):
                    o1[r, pl.ds(c * _LANE, _LANE)] = cbuf[r, pl.ds(c * _LANE, _LANE)]
                    o2[r, pl.ds(c * _LANE, _LANE)] = cbuf[r, pl.ds(D + c * _LANE, _LANE)]
            return carry

        lax.fori_loop(0, _CHUNK // rows_per_iter, rows, 0)

    fire_gather(0, ca, sga)

    def body(t, carry):
        g0 = 2 * t
        g1 = g0 + 1
        wait_gather(ca, sga)
        fire_gather(g1, cb, sgb)
        pl.when(t > 0)(lambda: wait_scatter(o1a, o2a, ssa))
        deinterleave(ca, o1a, o2a)
        fire_scatter(g0, o1a, o2a, ssa)
        wait_gather(cb, sgb)
        pl.when(t < n_chunks // 2 - 1)(lambda: fire_gather(g0 + 2, ca, sga))
        pl.when(t > 0)(lambda: wait_scatter(o1b, o2b, ssb))
        deinterleave(cb, o1b, o2b)
        fire_scatter(g1, o1b, o2b, ssb)
        return carry

    lax.fori_loop(0, n_chunks // 2, body, 0)
    wait_scatter(o1a, o2a, ssa)
    wait_scatter(o1b, o2b, ssb)


def kernel(tokens, embedding, embedding2):
    B, L = tokens.shape
    V, D = embedding.shape
    n_flat = B * L
    assert n_flat % _CHUNK == 0
    nw = _NC * _NS
    n_chunks_total = n_flat // _CHUNK
    assert n_chunks_total % nw == 0
    n_chunks = n_chunks_total // nw
    assert n_chunks % 2 == 0

    tok2d = tokens.astype(jnp.int32).reshape(n_chunks_total, _CHUNK)
    cat = jnp.concatenate([embedding, embedding2], axis=1)

    mesh = plsc.VectorSubcoreMesh(core_axis_name="c", subcore_axis_name="s")
    out_sds = jax.ShapeDtypeStruct((n_flat, D), jnp.float32)
    run = pl.kernel(
        functools.partial(_sc_body, n_chunks, D),
        mesh=mesh,
        out_type=[out_sds, out_sds],
        scratch_types=[
            pltpu.VMEM((n_chunks, _CHUNK), jnp.int32),
            pltpu.VMEM((_CHUNK, 2 * D), jnp.float32),
            pltpu.VMEM((_CHUNK, 2 * D), jnp.float32),
            pltpu.VMEM((_CHUNK, D), jnp.float32),
            pltpu.VMEM((_CHUNK, D), jnp.float32),
            pltpu.VMEM((_CHUNK, D), jnp.float32),
            pltpu.VMEM((_CHUNK, D), jnp.float32),
            pltpu.SemaphoreType.DMA,
            pltpu.SemaphoreType.DMA,
            pltpu.SemaphoreType.DMA,
            pltpu.SemaphoreType.DMA,
        ],
    )
    out1, out2 = run(tok2d, cat)
    return (out1.reshape(B, L, D), out2.reshape(B, L, D))


# speedup vs baseline: 6.5240x; 1.0622x over previous
import functools

import jax
import jax.numpy as jnp
from jax import lax
from jax.experimental import pallas as pl
from jax.experimental.pallas import tpu as pltpu, tpu_sc as plsc

_NC = 2
_NS = 16
_LANE = 16
_BB = 128


def _sc_body(B, L, D, tok_hbm, tabT_hbm, out_t,
             tabT_v, tok_v, obuf_a, obuf_b, sem_a, sem_b):
    c = lax.axis_index("c")
    s = lax.axis_index("s")
    wid = s * _NC + c
    b0 = wid * _BB

    pltpu.sync_copy(tabT_hbm.at[:, :], tabT_v)
    pltpu.sync_copy(tok_hbm.at[pl.ds(b0, _BB), :], tok_v)

    iota = lax.iota(jnp.int32, _LANE)
    ng = _BB // _LANE
    row_idx = [iota + g * _LANE for g in range(ng)]

    def splat(x):
        return jnp.full((_LANE,), x, jnp.int32)

    def compute(l, obuf):
        tokv = [plsc.load_gather(tok_v, [row_idx[g], splat(l)]) for g in range(ng)]

        def dloop(i, carry):
            for du in range(2):
                d = 2 * i + du
                for g in range(ng):
                    v = plsc.load_gather(tabT_v, [splat(d), tokv[g]])
                    obuf[d, pl.ds(g * _LANE, _LANE)] = v
            return carry

        lax.fori_loop(0, D // 2, dloop, 0)

    def fire(l, obuf, sem):
        pltpu.async_copy(obuf, out_t.at[l, :, pl.ds(b0, _BB)], sem)

    def drain(obuf, sem):
        pltpu.make_async_copy(obuf, out_t.at[0, :, pl.ds(0, _BB)], sem).wait()

    def lbody(t, carry):
        l0 = 2 * t
        pl.when(t > 0)(lambda: drain(obuf_a, sem_a))
        compute(l0, obuf_a)
        fire(l0, obuf_a, sem_a)
        pl.when(t > 0)(lambda: drain(obuf_b, sem_b))
        compute(l0 + 1, obuf_b)
        fire(l0 + 1, obuf_b, sem_b)
        return carry

    lax.fori_loop(0, L // 2, lbody, 0)
    drain(obuf_a, sem_a)
    drain(obuf_b, sem_b)


def kernel(tokens, embedding, embedding2):
    B, L = tokens.shape
    V, D = embedding.shape
    assert B == _BB * _NC * _NS and L % 2 == 0 and D % (2 * _LANE) == 0

    tok = tokens.astype(jnp.int32)

    mesh = plsc.VectorSubcoreMesh(core_axis_name="c", subcore_axis_name="s")
    run = pl.kernel(
        functools.partial(_sc_body, B, L, D),
        mesh=mesh,
        out_type=[jax.ShapeDtypeStruct((L, D, B), jnp.float32)],
        scratch_types=[
            pltpu.VMEM((D, V), jnp.float32),
            pltpu.VMEM((_BB, L), jnp.int32),
            pltpu.VMEM((D, _BB), jnp.float32),
            pltpu.VMEM((D, _BB), jnp.float32),
            pltpu.SemaphoreType.DMA,
            pltpu.SemaphoreType.DMA,
        ],
        compiler_params=pltpu.CompilerParams(needs_layout_passes=False),
    )
    (o1,) = run(tok, embedding.T)
    (o2,) = run(tok, embedding2.T)
    return (o1.transpose(2, 0, 1), o2.transpose(2, 0, 1))
